# Initial kernel scaffold; baseline (speedup 1.0000x reference)
import jax
import jax.numpy as jnp
from jax.experimental import pallas as pl


def kernel(tokens, table):
    raise NotImplementedError("write your pallas kernel here")


# speedup vs baseline: 4.7569x; 4.7569x over previous
import functools
import math

import jax
import jax.numpy as jnp
from jax import lax
from jax.experimental import pallas as pl
from jax.experimental.pallas import tpu as pltpu
from jax.experimental.pallas import tpu_sc as plsc


def _scale_table(table, scale):
    V, D = table.shape
    blk = 2000
    def body(t_ref, o_ref):
        o_ref[...] = t_ref[...] * scale
    return pl.pallas_call(
        body,
        grid=(V // blk,),
        in_specs=[pl.BlockSpec((blk, D), lambda i: (i, 0))],
        out_specs=pl.BlockSpec((blk, D), lambda i: (i, 0)),
        out_shape=jax.ShapeDtypeStruct((V, D), jnp.float32),
    )(table)


@functools.partial(jax.jit, static_argnames=("B", "D"))
def _sc_gather(idx, table, B, D):
    info = plsc.get_sparse_core_info()
    NC, NS = info.num_cores, info.num_subcores
    NW = NC * NS
    b_per_w = B // NW
    C = 128
    n_chunks = b_per_w // C

    mesh = plsc.VectorSubcoreMesh(core_axis_name="c", subcore_axis_name="s")

    @functools.partial(
        pl.kernel,
        mesh=mesh,
        out_type=jax.ShapeDtypeStruct((B, D), jnp.float32),
        scratch_types=[
            pltpu.VMEM((1, C), jnp.int32),
            pltpu.VMEM((C, D), jnp.float32),
            pltpu.SemaphoreType.DMA,
        ],
    )
    def k(idx_hbm, table_hbm, out_hbm, idx_v, rows_v, sem):
        wid = lax.axis_index("s") * NC + lax.axis_index("c")
        base = wid * b_per_w

        def body(i, carry):
            off = base + i * C
            pltpu.sync_copy(idx_hbm.at[pl.ds(off, C)], idx_v.at[0])
            pltpu.async_copy(table_hbm.at[idx_v.at[0]], rows_v, sem).wait()
            pltpu.sync_copy(rows_v, out_hbm.at[pl.ds(off, C)])
            return carry

        lax.fori_loop(0, n_chunks, body, 0)

    return k(idx, table)


def kernel(tokens, table):
    Bt, T = tokens.shape
    V, D = table.shape
    B = Bt * T
    scaled = _scale_table(table, math.sqrt(D))
    idx = tokens.reshape(B).astype(jnp.int32)
    out = _sc_gather(idx, scaled, B=B, D=D)
    return out.reshape(Bt, T, D)


# speedup vs baseline: 7.9431x; 1.6698x over previous
import functools
import math

import jax
import jax.numpy as jnp
from jax import lax
from jax.experimental import pallas as pl
from jax.experimental.pallas import tpu as pltpu
from jax.experimental.pallas import tpu_sc as plsc


def _scale_table(table, scale):
    V, D = table.shape
    blk = 2000
    def body(t_ref, o_ref):
        o_ref[...] = t_ref[...] * scale
    return pl.pallas_call(
        body,
        grid=(V // blk,),
        in_specs=[pl.BlockSpec((blk, D), lambda i: (i, 0))],
        out_specs=pl.BlockSpec((blk, D), lambda i: (i, 0)),
        out_shape=jax.ShapeDtypeStruct((V, D), jnp.float32),
    )(table)


@functools.partial(jax.jit, static_argnames=("B", "D"))
def _sc_gather(idx2d, table, B, D):
    info = plsc.get_sparse_core_info()
    NC, NS = info.num_cores, info.num_subcores
    NW = NC * NS
    b_per_w = B // NW
    C = 128
    n_chunks = b_per_w // C
    NBUF = 4

    mesh = plsc.VectorSubcoreMesh(core_axis_name="c", subcore_axis_name="s")

    @functools.partial(
        pl.kernel,
        mesh=mesh,
        out_type=jax.ShapeDtypeStruct((B, D), jnp.float32),
        scratch_types=(
            [pltpu.VMEM((n_chunks, C), jnp.int32)]
            + [pltpu.VMEM((C, D), jnp.float32) for _ in range(NBUF)]
            + [pltpu.SemaphoreType.DMA for _ in range(2 * NBUF)]
        ),
    )
    def k(idx_hbm, table_hbm, out_hbm, idx_v, r0, r1, r2, r3,
          g0, g1, g2, g3, s0, s1, s2, s3):
        rows = (r0, r1, r2, r3)
        gsem = (g0, g1, g2, g3)
        ssem = (s0, s1, s2, s3)
        wid = lax.axis_index("s") * NC + lax.axis_index("c")
        out_base = wid * b_per_w

        pltpu.sync_copy(idx_hbm.at[pl.ds(wid * n_chunks, n_chunks)], idx_v)

        def gather_start(g, j):
            pltpu.async_copy(table_hbm.at[idx_v.at[g]], rows[j], gsem[j])

        def gather_wait(j):
            pltpu.make_async_copy(
                table_hbm.at[idx_v.at[0]], rows[j], gsem[j]).wait()

        def scatter_start(g, j):
            pltpu.async_copy(
                rows[j], out_hbm.at[pl.ds(out_base + g * C, C)], ssem[j])

        def scatter_wait(j):
            pltpu.make_async_copy(
                rows[j], out_hbm.at[pl.ds(out_base, C)], ssem[j]).wait()

        def outer(o, carry):
            gbase = o * NBUF
            for j in range(NBUF):
                g = gbase + j
                @pl.when(o > 0)
                def _(j=j):
                    scatter_wait(j)
                gather_start(g, j)
                jp = (j + NBUF - 1) % NBUF
                if j == 0:
                    @pl.when(o > 0)
                    def _(g=g, jp=jp):
                        gather_wait(jp)
                        scatter_start(g - 1, jp)
                else:
                    gather_wait(jp)
                    scatter_start(g - 1, jp)
            return carry

        lax.fori_loop(0, n_chunks // NBUF, outer, 0)

        last = n_chunks - 1
        gather_wait(last % NBUF)
        scatter_start(last, last % NBUF)
        for j in range(NBUF):
            scatter_wait(j)

    return k(idx2d, table)


def kernel(tokens, table):
    Bt, T = tokens.shape
    V, D = table.shape
    B = Bt * T
    C = 128
    scaled = _scale_table(table, math.sqrt(D))
    idx2d = tokens.reshape(B // C, C).astype(jnp.int32)
    out = _sc_gather(idx2d, scaled, B=B, D=D)
    return out.reshape(Bt, T, D)


# speedup vs baseline: 8.3086x; 1.0460x over previous
import functools
import math

import jax
import jax.numpy as jnp
from jax import lax
from jax.experimental import pallas as pl
from jax.experimental.pallas import tpu as pltpu
from jax.experimental.pallas import tpu_sc as plsc


def _scale_table(table, scale):
    V, D = table.shape
    blk = 4000
    def body(t_ref, o_ref):
        o_ref[...] = t_ref[...] * scale
    return pl.pallas_call(
        body,
        grid=(V // blk,),
        in_specs=[pl.BlockSpec((blk, D), lambda i: (i, 0))],
        out_specs=pl.BlockSpec((blk, D), lambda i: (i, 0)),
        out_shape=jax.ShapeDtypeStruct((V, D), jnp.float32),
    )(table)


@functools.partial(jax.jit, static_argnames=("B", "D"))
def _sc_gather(idx2d, table, B, D):
    info = plsc.get_sparse_core_info()
    NC, NS = info.num_cores, info.num_subcores
    NW = NC * NS
    b_per_w = B // NW
    C = 128
    n_chunks = b_per_w // C
    NBUF = 4

    mesh = plsc.VectorSubcoreMesh(core_axis_name="c", subcore_axis_name="s")

    @functools.partial(
        pl.kernel,
        mesh=mesh,
        out_type=jax.ShapeDtypeStruct((B, D), jnp.float32),
        scratch_types=(
            [pltpu.VMEM((n_chunks, C), jnp.int32)]
            + [pltpu.VMEM((C, D), jnp.float32) for _ in range(NBUF)]
            + [pltpu.SemaphoreType.DMA for _ in range(2 * NBUF)]
        ),
    )
    def k(idx_hbm, table_hbm, out_hbm, idx_v, r0, r1, r2, r3,
          g0, g1, g2, g3, s0, s1, s2, s3):
        rows = (r0, r1, r2, r3)
        gsem = (g0, g1, g2, g3)
        ssem = (s0, s1, s2, s3)
        wid = lax.axis_index("s") * NC + lax.axis_index("c")
        out_base = wid * b_per_w

        pltpu.sync_copy(idx_hbm.at[pl.ds(wid * n_chunks, n_chunks)], idx_v)

        def gather_start(g, j):
            pltpu.async_copy(table_hbm.at[idx_v.at[g]], rows[j], gsem[j])

        def gather_wait(j):
            pltpu.make_async_copy(
                table_hbm.at[idx_v.at[0]], rows[j], gsem[j]).wait()

        def scatter_start(g, j):
            pltpu.async_copy(
                rows[j], out_hbm.at[pl.ds(out_base + g * C, C)], ssem[j])

        def scatter_wait(j):
            pltpu.make_async_copy(
                rows[j], out_hbm.at[pl.ds(out_base, C)], ssem[j]).wait()

        LAG = 2

        def outer(o, carry):
            gbase = o * NBUF
            for j in range(NBUF):
                g = gbase + j
                @pl.when(o > 0)
                def _(j=j):
                    scatter_wait(j)
                gather_start(g, j)
                jp = (j + NBUF - LAG) % NBUF
                if j < LAG:
                    @pl.when(o > 0)
                    def _(g=g, jp=jp):
                        gather_wait(jp)
                        scatter_start(g - LAG, jp)
                else:
                    gather_wait(jp)
                    scatter_start(g - LAG, jp)
            return carry

        lax.fori_loop(0, n_chunks // NBUF, outer, 0)

        for t in range(LAG):
            g = n_chunks - LAG + t
            gather_wait(g % NBUF)
            scatter_start(g, g % NBUF)
        for j in range(NBUF):
            scatter_wait(j)

    return k(idx2d, table)


def kernel(tokens, table):
    Bt, T = tokens.shape
    V, D = table.shape
    B = Bt * T
    C = 128
    scaled = _scale_table(table, math.sqrt(D))
    idx2d = tokens.reshape(B // C, C).astype(jnp.int32)
    out = _sc_gather(idx2d, scaled, B=B, D=D)
    return out.reshape(Bt, T, D)
